# baseline (speedup 1.0000x reference)
import functools

import jax
import jax.numpy as jnp
from jax import lax
from jax.experimental import pallas as pl
from jax.experimental.pallas import tpu as pltpu
from jax.experimental.pallas import tpu_sc as plsc

B = 16
D = 64
HW = 1024
N = 1024
BETA = 0.25

NC = 2
NS = 16
L = 16
DPT = D // 2


def _tc_body(z_ref, w_ref, idx_ref, acc_ref):
    b = pl.program_id(0)
    zc = z_ref[0]
    w = w_ref[...]
    wsq = jnp.sum(w * w, axis=1, keepdims=True)
    dots2 = jax.lax.dot_general(
        -2.0 * w, zc, (((1,), (0,)), ((), ())),
        preferred_element_type=jnp.float32)
    dist_t = wsq + dots2
    min_d = jnp.min(dist_t, axis=0, keepdims=True)
    iota_t = jax.lax.broadcasted_iota(jnp.int32, (N, HW), 0)
    idx = jnp.min(jnp.where(dist_t == min_d, iota_t, N), axis=0)
    idx_ref[0, 0] = idx
    val = jnp.sum(min_d) + jnp.sum(zc * zc)

    @pl.when(b == 0)
    def _():
        acc_ref[0, 0] = val

    @pl.when(b > 0)
    def _():
        acc_ref[0, 0] += val


def _sc_body(wt_hbm, idx_hbm, zq_hbm, div_hbm,
             idx_v, wt_v, out_v, bm_v, div_v):
    wid = lax.axis_index("s") * NC + lax.axis_index("c")
    b = wid // 2
    h = wid % 2
    d0 = h * DPT
    pltpu.sync_copy(idx_hbm.at[b], idx_v)
    pltpu.sync_copy(wt_hbm.at[pl.ds(d0 * N, DPT * N)], wt_v)

    def gbody(i, c):
        idx16 = idx_v[pl.ds(i * L, L)]
        for r in range(DPT):
            fidx = idx16 + jnp.int32(r * N)
            out_v[pl.ds(r * HW + i * L, L)] = plsc.load_gather(wt_v, [fidx])
        return c

    lax.fori_loop(0, HW // L, gbody, 0)
    pltpu.sync_copy(out_v, zq_hbm.at[b, pl.ds(d0 * HW, DPT * HW)])


    @pl.when(h == 0)
    def _():
        def zbody(i, c):
            bm_v[pl.ds(i * L, L)] = jnp.zeros((L,), jnp.float32)
            return c

        lax.fori_loop(0, N // L, zbody, 0)

        def sbody(i, c):
            idx16 = idx_v[pl.ds(i * L, L)]
            plsc.store_scatter(bm_v, [idx16], jnp.ones((L,), jnp.float32))
            return c

        lax.fori_loop(0, HW // L, sbody, 0)

        def cbody(i, a):
            return a + bm_v[pl.ds(i * L, L)]

        acc16 = lax.fori_loop(0, N // L, cbody, jnp.zeros((L,), jnp.float32))
        cnt = jnp.sum(acc16)
        div_v[...] = jnp.full((L,), cnt, jnp.float32)
        pltpu.sync_copy(div_v, div_hbm.at[b])


def kernel(z, weight):
    zr = z.reshape(B, D, HW)
    idx, acc = pl.pallas_call(
        _tc_body,
        grid=(B,),
        in_specs=[
            pl.BlockSpec((1, D, HW), lambda b: (b, 0, 0)),
            pl.BlockSpec((N, D), lambda b: (0, 0)),
        ],
        out_specs=[
            pl.BlockSpec((1, 1, HW), lambda b: (b, 0, 0)),
            pl.BlockSpec(memory_space=pltpu.SMEM),
        ],
        out_shape=[
            jax.ShapeDtypeStruct((B, 1, HW), jnp.int32),
            jax.ShapeDtypeStruct((1, 1), jnp.float32),
        ],
        compiler_params=pltpu.CompilerParams(
            dimension_semantics=("arbitrary",),
        ),
    )(zr, weight)

    mesh = plsc.VectorSubcoreMesh(core_axis_name="c", subcore_axis_name="s")
    sc_call = pl.kernel(
        _sc_body,
        out_type=[
            jax.ShapeDtypeStruct((B, D * HW), jnp.float32),
            jax.ShapeDtypeStruct((B, L), jnp.float32),
        ],
        mesh=mesh,
        compiler_params=pltpu.CompilerParams(needs_layout_passes=False),
        scratch_types=[
            pltpu.VMEM((HW,), jnp.int32),
            pltpu.VMEM((DPT * N,), jnp.float32),
            pltpu.VMEM((DPT * HW,), jnp.float32),
            pltpu.VMEM((N,), jnp.float32),
            pltpu.VMEM((L,), jnp.float32),
        ],
    )
    wt = weight.T.reshape(D * N)
    zq, divv = sc_call(wt, idx.reshape(B, HW))

    z_q_out = zq.reshape(B, D, 32, 32)
    index = idx.reshape(B, 32, 32)
    loss = acc[0, 0] * ((1.0 + BETA) / (B * HW * D))
    diversity = jnp.sum(divv[:, 0]) / (B * HW)
    return z_q_out, index, loss, diversity


# speedup vs baseline: 1.6961x; 1.6961x over previous
import jax
import jax.numpy as jnp
from jax.experimental import pallas as pl
from jax.experimental.pallas import tpu as pltpu

B = 16
D = 64
HW = 1024
N = 1024
BETA = 0.25


def _vq_body(z_ref, w_ref, zq_ref, idx_ref, acc_ref, div_ref):
    b = pl.program_id(0)
    zc = z_ref[0]
    w = w_ref[...]
    wsq = jnp.sum(w * w, axis=1, keepdims=True)
    dots2 = jax.lax.dot_general(
        -2.0 * w, zc, (((1,), (0,)), ((), ())),
        preferred_element_type=jnp.float32)
    dist_t = wsq + dots2
    min_d = jnp.min(dist_t, axis=0, keepdims=True)
    iota_t = jax.lax.broadcasted_iota(jnp.int32, (N, HW), 0)
    keys = jax.lax.bitcast_convert_type(
        iota_t + jnp.int32(0x4B000000), jnp.float32)
    sentinel = jax.lax.bitcast_convert_type(
        jnp.int32(0x4B000000 + N), jnp.float32)
    keymin = jnp.min(jnp.where(dist_t == min_d, keys, sentinel),
                     axis=0, keepdims=True)
    idx = (jax.lax.bitcast_convert_type(keymin, jnp.int32)
           - jnp.int32(0x4B000000))[0]
    idx_ref[0, 0] = idx
    ohf = (keys == keymin).astype(jnp.float32)
    cnts = jax.lax.dot_general(
        ohf, jnp.ones((HW, 128), jnp.float32),
        (((1,), (0,)), ((), ())),
        preferred_element_type=jnp.float32)
    usedf = (cnts[:, 0:1] > 0.0).astype(jnp.float32)
    dval = jnp.sum(usedf)
    zq = jax.lax.dot_general(
        w, ohf, (((0,), (0,)), ((), ())),
        preferred_element_type=jnp.float32)
    zq_ref[0] = zq
    val = jnp.sum(min_d) + jnp.sum(zc * zc)

    @pl.when(b == 0)
    def _():
        acc_ref[0, 0] = val
        div_ref[0, 0] = dval

    @pl.when(b > 0)
    def _():
        acc_ref[0, 0] += val
        div_ref[0, 0] += dval


def kernel(z, weight):
    zr = z.reshape(B, D, HW)
    zq, idx, acc, div = pl.pallas_call(
        _vq_body,
        grid=(B,),
        in_specs=[
            pl.BlockSpec((1, D, HW), lambda b: (b, 0, 0)),
            pl.BlockSpec((N, D), lambda b: (0, 0)),
        ],
        out_specs=[
            pl.BlockSpec((1, D, HW), lambda b: (b, 0, 0)),
            pl.BlockSpec((1, 1, HW), lambda b: (b, 0, 0)),
            pl.BlockSpec(memory_space=pltpu.SMEM),
            pl.BlockSpec(memory_space=pltpu.SMEM),
        ],
        out_shape=[
            jax.ShapeDtypeStruct((B, D, HW), jnp.float32),
            jax.ShapeDtypeStruct((B, 1, HW), jnp.int32),
            jax.ShapeDtypeStruct((1, 1), jnp.float32),
            jax.ShapeDtypeStruct((1, 1), jnp.float32),
        ],
        compiler_params=pltpu.CompilerParams(
            dimension_semantics=("arbitrary",),
        ),
    )(zr, weight)
    z_q_out = zq.reshape(B, D, 32, 32)
    index = idx.reshape(B, 32, 32)
    loss = acc[0, 0] * ((1.0 + BETA) / (B * HW * D))
    diversity = div[0, 0] / (B * HW)
    return z_q_out, index, loss, diversity


# speedup vs baseline: 1.8005x; 1.0616x over previous
import jax
import jax.numpy as jnp
from jax.experimental import pallas as pl
from jax.experimental.pallas import tpu as pltpu

B = 16
D = 64
HW = 1024
N = 1024
BETA = 0.25
BPG = 2


def _vq_body(z_ref, w_ref, zq_ref, idx_ref, acc_ref, div_ref):
    g = pl.program_id(0)
    w = w_ref[...]
    wsq = jnp.sum(w * w, axis=1, keepdims=True)
    w2 = -2.0 * w
    vals = []
    dvals = []
    for i in range(BPG):
        zc = z_ref[i]
        dots2 = jax.lax.dot_general(
            w2, zc, (((1,), (0,)), ((), ())),
            preferred_element_type=jnp.float32)
        dist_t = wsq + dots2
        min_d = jnp.min(dist_t, axis=0, keepdims=True)
        iota_t = jax.lax.broadcasted_iota(jnp.int32, (N, HW), 0)
        keys = jax.lax.bitcast_convert_type(
            iota_t + jnp.int32(0x4B000000), jnp.float32)
        sentinel = jax.lax.bitcast_convert_type(
            jnp.int32(0x4B000000 + N), jnp.float32)
        keymin = jnp.min(jnp.where(dist_t == min_d, keys, sentinel),
                         axis=0, keepdims=True)
        idx = (jax.lax.bitcast_convert_type(keymin, jnp.int32)
               - jnp.int32(0x4B000000))[0]
        idx_ref[i, 0] = idx
        ohf = (keys == keymin).astype(jnp.float32)
        cnts = jax.lax.dot_general(
            ohf, jnp.ones((HW, 128), jnp.float32),
            (((1,), (0,)), ((), ())),
            preferred_element_type=jnp.float32)
        usedf = (cnts[:, 0:1] > 0.0).astype(jnp.float32)
        dvals.append(jnp.sum(usedf))
        zq = jax.lax.dot_general(
            w, ohf, (((0,), (0,)), ((), ())),
            preferred_element_type=jnp.float32)
        zq_ref[i] = zq
        vals.append(jnp.sum(min_d) + jnp.sum(zc * zc))
    val = vals[0] + vals[1]
    dval = dvals[0] + dvals[1]

    @pl.when(g == 0)
    def _():
        acc_ref[0, 0] = val
        div_ref[0, 0] = dval

    @pl.when(g > 0)
    def _():
        acc_ref[0, 0] += val
        div_ref[0, 0] += dval


def kernel(z, weight):
    zr = z.reshape(B, D, HW)
    zq, idx, acc, div = pl.pallas_call(
        _vq_body,
        grid=(B // BPG,),
        in_specs=[
            pl.BlockSpec((BPG, D, HW), lambda b: (b, 0, 0)),
            pl.BlockSpec((N, D), lambda b: (0, 0)),
        ],
        out_specs=[
            pl.BlockSpec((BPG, D, HW), lambda b: (b, 0, 0)),
            pl.BlockSpec((BPG, 1, HW), lambda b: (b, 0, 0)),
            pl.BlockSpec(memory_space=pltpu.SMEM),
            pl.BlockSpec(memory_space=pltpu.SMEM),
        ],
        out_shape=[
            jax.ShapeDtypeStruct((B, D, HW), jnp.float32),
            jax.ShapeDtypeStruct((B, 1, HW), jnp.int32),
            jax.ShapeDtypeStruct((1, 1), jnp.float32),
            jax.ShapeDtypeStruct((1, 1), jnp.float32),
        ],
        compiler_params=pltpu.CompilerParams(
            dimension_semantics=("arbitrary",),
        ),
    )(zr, weight)
    z_q_out = zq.reshape(B, D, 32, 32)
    index = idx.reshape(B, 32, 32)
    loss = acc[0, 0] * ((1.0 + BETA) / (B * HW * D))
    diversity = div[0, 0] / (B * HW)
    return z_q_out, index, loss, diversity


# speedup vs baseline: 1.8665x; 1.0367x over previous
import jax
import jax.numpy as jnp
from jax.experimental import pallas as pl
from jax.experimental.pallas import tpu as pltpu

B = 16
D = 64
HW = 1024
N = 1024
BETA = 0.25
BPG = 4


def _vq_body(z_ref, w_ref, zq_ref, idx_ref, acc_ref, div_ref):
    g = pl.program_id(0)
    w = w_ref[...]
    wsq = jnp.sum(w * w, axis=1, keepdims=True)
    w2 = -2.0 * w
    vals = []
    dvals = []
    for i in range(BPG):
        zc = z_ref[i]
        dots2 = jax.lax.dot_general(
            w2, zc, (((1,), (0,)), ((), ())),
            preferred_element_type=jnp.float32)
        dist_t = wsq + dots2
        min_d = jnp.min(dist_t, axis=0, keepdims=True)
        iota_t = jax.lax.broadcasted_iota(jnp.int32, (N, HW), 0)
        keys = jax.lax.bitcast_convert_type(
            iota_t + jnp.int32(0x4B000000), jnp.float32)
        sentinel = jax.lax.bitcast_convert_type(
            jnp.int32(0x4B000000 + N), jnp.float32)
        keymin = jnp.min(jnp.where(dist_t == min_d, keys, sentinel),
                         axis=0, keepdims=True)
        idx = (jax.lax.bitcast_convert_type(keymin, jnp.int32)
               - jnp.int32(0x4B000000))[0]
        idx_ref[i, 0] = idx
        ohf = (keys == keymin).astype(jnp.float32)
        cnts = jax.lax.dot_general(
            ohf, jnp.ones((HW, 128), jnp.float32),
            (((1,), (0,)), ((), ())),
            preferred_element_type=jnp.float32)
        usedf = (cnts[:, 0:1] > 0.0).astype(jnp.float32)
        dvals.append(jnp.sum(usedf))
        zq = jax.lax.dot_general(
            w, ohf, (((0,), (0,)), ((), ())),
            preferred_element_type=jnp.float32)
        zq_ref[i] = zq
        vals.append(jnp.sum(min_d) + jnp.sum(zc * zc))
    val = sum(vals)
    dval = sum(dvals)

    @pl.when(g == 0)
    def _():
        acc_ref[0, 0] = val
        div_ref[0, 0] = dval

    @pl.when(g > 0)
    def _():
        acc_ref[0, 0] += val
        div_ref[0, 0] += dval


def kernel(z, weight):
    zr = z.reshape(B, D, HW)
    zq, idx, acc, div = pl.pallas_call(
        _vq_body,
        grid=(B // BPG,),
        in_specs=[
            pl.BlockSpec((BPG, D, HW), lambda b: (b, 0, 0)),
            pl.BlockSpec((N, D), lambda b: (0, 0)),
        ],
        out_specs=[
            pl.BlockSpec((BPG, D, HW), lambda b: (b, 0, 0)),
            pl.BlockSpec((BPG, 1, HW), lambda b: (b, 0, 0)),
            pl.BlockSpec(memory_space=pltpu.SMEM),
            pl.BlockSpec(memory_space=pltpu.SMEM),
        ],
        out_shape=[
            jax.ShapeDtypeStruct((B, D, HW), jnp.float32),
            jax.ShapeDtypeStruct((B, 1, HW), jnp.int32),
            jax.ShapeDtypeStruct((1, 1), jnp.float32),
            jax.ShapeDtypeStruct((1, 1), jnp.float32),
        ],
        compiler_params=pltpu.CompilerParams(
            dimension_semantics=("arbitrary",),
        ),
    )(zr, weight)
    z_q_out = zq.reshape(B, D, 32, 32)
    index = idx.reshape(B, 32, 32)
    loss = acc[0, 0] * ((1.0 + BETA) / (B * HW * D))
    diversity = div[0, 0] / (B * HW)
    return z_q_out, index, loss, diversity


# speedup vs baseline: 1.8956x; 1.0156x over previous
import jax
import jax.numpy as jnp
from jax.experimental import pallas as pl
from jax.experimental.pallas import tpu as pltpu

B = 16
D = 64
HW = 1024
N = 1024
BETA = 0.25
BPG = 8


def _vq_body(z_ref, w_ref, zq_ref, idx_ref, acc_ref, div_ref):
    g = pl.program_id(0)
    w = w_ref[...]
    wsq = jnp.sum(w * w, axis=1, keepdims=True)
    w2 = -2.0 * w
    vals = []
    dvals = []
    for i in range(BPG):
        zc = z_ref[i]
        dots2 = jax.lax.dot_general(
            w2, zc, (((1,), (0,)), ((), ())),
            preferred_element_type=jnp.float32)
        dist_t = wsq + dots2
        min_d = jnp.min(dist_t, axis=0, keepdims=True)
        iota_t = jax.lax.broadcasted_iota(jnp.int32, (N, HW), 0)
        keys = jax.lax.bitcast_convert_type(
            iota_t + jnp.int32(0x4B000000), jnp.float32)
        sentinel = jax.lax.bitcast_convert_type(
            jnp.int32(0x4B000000 + N), jnp.float32)
        keymin = jnp.min(jnp.where(dist_t == min_d, keys, sentinel),
                         axis=0, keepdims=True)
        idx = (jax.lax.bitcast_convert_type(keymin, jnp.int32)
               - jnp.int32(0x4B000000))[0]
        idx_ref[i, 0] = idx
        ohf = (keys == keymin).astype(jnp.float32)
        cnts = jax.lax.dot_general(
            ohf, jnp.ones((HW, 128), jnp.float32),
            (((1,), (0,)), ((), ())),
            preferred_element_type=jnp.float32)
        usedf = (cnts[:, 0:1] > 0.0).astype(jnp.float32)
        dvals.append(jnp.sum(usedf))
        zq = jax.lax.dot_general(
            w, ohf, (((0,), (0,)), ((), ())),
            preferred_element_type=jnp.float32)
        zq_ref[i] = zq
        vals.append(jnp.sum(min_d) + jnp.sum(zc * zc))
    val = sum(vals)
    dval = sum(dvals)

    @pl.when(g == 0)
    def _():
        acc_ref[0, 0] = val
        div_ref[0, 0] = dval

    @pl.when(g > 0)
    def _():
        acc_ref[0, 0] += val
        div_ref[0, 0] += dval


def kernel(z, weight):
    zr = z.reshape(B, D, HW)
    zq, idx, acc, div = pl.pallas_call(
        _vq_body,
        grid=(B // BPG,),
        in_specs=[
            pl.BlockSpec((BPG, D, HW), lambda b: (b, 0, 0)),
            pl.BlockSpec((N, D), lambda b: (0, 0)),
        ],
        out_specs=[
            pl.BlockSpec((BPG, D, HW), lambda b: (b, 0, 0)),
            pl.BlockSpec((BPG, 1, HW), lambda b: (b, 0, 0)),
            pl.BlockSpec(memory_space=pltpu.SMEM),
            pl.BlockSpec(memory_space=pltpu.SMEM),
        ],
        out_shape=[
            jax.ShapeDtypeStruct((B, D, HW), jnp.float32),
            jax.ShapeDtypeStruct((B, 1, HW), jnp.int32),
            jax.ShapeDtypeStruct((1, 1), jnp.float32),
            jax.ShapeDtypeStruct((1, 1), jnp.float32),
        ],
        compiler_params=pltpu.CompilerParams(
            dimension_semantics=("arbitrary",),
        ),
    )(zr, weight)
    z_q_out = zq.reshape(B, D, 32, 32)
    index = idx.reshape(B, 32, 32)
    loss = acc[0, 0] * ((1.0 + BETA) / (B * HW * D))
    diversity = div[0, 0] / (B * HW)
    return z_q_out, index, loss, diversity
